# Initial kernel scaffold; baseline (speedup 1.0000x reference)
import jax
import jax.numpy as jnp
from jax.experimental import pallas as pl


def kernel(x, edge_attr, params, edge_index):
    raise NotImplementedError("write your pallas kernel here")


# speedup vs baseline: 1.8097x; 1.8097x over previous
import functools

import jax
import jax.numpy as jnp
from jax import lax
from jax.experimental import pallas as pl
from jax.experimental.pallas import tpu as pltpu
from jax.experimental.pallas import tpu_sc as plsc

_NC, _NS = 2, 16
_NW = _NC * _NS
_CHUNK = 80


def _make_gather(N, E, D):
    per_w = E // _NW
    n_chunks = per_w // _CHUNK
    mesh = plsc.VectorSubcoreMesh(core_axis_name="c", subcore_axis_name="s")

    def body(h_hbm, src_hbm, dst_hbm, hs_hbm, hd_hbm,
             idx_s, idx_d, rows_s, rows_d, sem_s, sem_d):
        wid = lax.axis_index("s") * _NC + lax.axis_index("c")
        base = wid * per_w

        def chunk(i, carry):
            st = base + i * _CHUNK
            pltpu.sync_copy(src_hbm.at[pl.ds(st, _CHUNK)], idx_s)
            pltpu.sync_copy(dst_hbm.at[pl.ds(st, _CHUNK)], idx_d)
            cps = pltpu.async_copy(h_hbm.at[idx_s], rows_s, sem_s)
            cpd = pltpu.async_copy(h_hbm.at[idx_d], rows_d, sem_d)
            cps.wait()
            cpd.wait()
            pltpu.sync_copy(rows_s, hs_hbm.at[pl.ds(st, _CHUNK)])
            pltpu.sync_copy(rows_d, hd_hbm.at[pl.ds(st, _CHUNK)])
            return carry

        lax.fori_loop(0, n_chunks, chunk, 0)

    return pl.kernel(
        body,
        out_type=(jax.ShapeDtypeStruct((E, D), jnp.float32),
                  jax.ShapeDtypeStruct((E, D), jnp.float32)),
        mesh=mesh,
        scratch_types=[
            pltpu.VMEM((_CHUNK,), jnp.int32),
            pltpu.VMEM((_CHUNK,), jnp.int32),
            pltpu.VMEM((_CHUNK, D), jnp.float32),
            pltpu.VMEM((_CHUNK, D), jnp.float32),
            pltpu.SemaphoreType.DMA,
            pltpu.SemaphoreType.DMA,
        ],
        compiler_params=pltpu.CompilerParams(use_tc_tiling_on_sc=False),
    )


def _make_scatter(N, E, D):
    per_w = E // _NW
    n_chunks = per_w // _CHUNK
    rows_per_s = N // _NS
    mesh = plsc.VectorSubcoreMesh(core_axis_name="c", subcore_axis_name="s")

    def body(e_hbm, dst_hbm, zeros_hbm, out_hbm, acc_sh, idx_v, rows_v):
        c = lax.axis_index("c")
        s = lax.axis_index("s")
        wid = s * _NC + c
        base = wid * per_w
        pltpu.sync_copy(zeros_hbm.at[pl.ds(s * rows_per_s, rows_per_s)],
                        acc_sh.at[pl.ds(s * rows_per_s, rows_per_s)])
        plsc.subcore_barrier()

        def chunk(i, carry):
            st = base + i * _CHUNK
            pltpu.sync_copy(dst_hbm.at[pl.ds(st, _CHUNK)], idx_v)
            pltpu.sync_copy(e_hbm.at[pl.ds(st, _CHUNK)], rows_v)
            pltpu.sync_copy(rows_v, acc_sh.at[idx_v], add=True)
            return carry

        lax.fori_loop(0, n_chunks, chunk, 0)
        plsc.subcore_barrier()
        pltpu.sync_copy(acc_sh.at[pl.ds(s * rows_per_s, rows_per_s)],
                        out_hbm.at[c, pl.ds(s * rows_per_s, rows_per_s)])

    return pl.kernel(
        body,
        out_type=jax.ShapeDtypeStruct((_NC, N, D), jnp.float32),
        mesh=mesh,
        scratch_types=[
            pltpu.VMEM_SHARED((N, D), jnp.float32),
            pltpu.VMEM((_CHUNK,), jnp.int32),
            pltpu.VMEM((_CHUNK, D), jnp.float32),
        ],
        compiler_params=pltpu.CompilerParams(use_tc_tiling_on_sc=False),
    )


def _dot(a, b):
    return jnp.dot(a, b, preferred_element_type=jnp.float32)


def _node_enc_body(x_ref, w1, b1, w2, b2, o_ref):
    z = jnp.maximum(_dot(x_ref[...], w1[...]) + b1[...], 0.0)
    o_ref[...] = _dot(z, w2[...]) + b2[...]


def _node_upd_body(h_ref, acc_ref, w1, b1, w2, b2, o_ref):
    agg = acc_ref[0] + acc_ref[1]
    hin = jnp.concatenate([h_ref[...], agg], axis=1)
    z = jnp.maximum(_dot(hin, w1[...]) + b1[...], 0.0)
    o_ref[...] = _dot(z, w2[...]) + b2[...]


def _edge0_body(ea, hs, hd, ew1, eb1, ew2, eb2, w1, b1, w2, b2, o_ref):
    e0 = _dot(jnp.maximum(_dot(ea[...], ew1[...]) + eb1[...], 0.0),
              ew2[...]) + eb2[...]
    ein = jnp.concatenate([hs[...], hd[...], e0], axis=1)
    z = jnp.maximum(_dot(ein, w1[...]) + b1[...], 0.0)
    o_ref[...] = _dot(z, w2[...]) + b2[...]


def _edgek_body(e, hs, hd, w1, b1, w2, b2, o_ref):
    ein = jnp.concatenate([hs[...], hd[...], e[...]], axis=1)
    z = jnp.maximum(_dot(ein, w1[...]) + b1[...], 0.0)
    o_ref[...] = _dot(z, w2[...]) + b2[...]


def _head_body(e, hs, hd, w1, b1, w2, b2, w3, b3, o_ref):
    ein = jnp.concatenate([hs[...], hd[...], e[...]], axis=1)
    z = jnp.maximum(_dot(ein, w1[...]) + b1[...], 0.0)
    z2 = jnp.maximum(_dot(z, w2[...]) + b2[...], 0.0)
    o_ref[...] = _dot(z2, w3[...]) + b3[...]


def _full(shape):
    return pl.BlockSpec(shape, lambda i: (0,) * len(shape))


def _rows(be, d):
    return pl.BlockSpec((be, d), lambda i: (i, 0))


def _edge_call(body, grid, in_specs, out_specs, out_shape):
    return pl.pallas_call(
        body, grid=(grid,), in_specs=in_specs, out_specs=out_specs,
        out_shape=out_shape,
        compiler_params=pltpu.CompilerParams(
            vmem_limit_bytes=100 * 1024 * 1024))


_BE = 6400


def kernel(x, edge_attr, params, edge_index):
    N, ND = x.shape
    E, ED = edge_attr.shape
    H = params['node_enc'][-1][0].shape[1]
    assert E % (_NW * _CHUNK) == 0 and N % _NS == 0 and E % _BE == 0
    grid = E // _BE

    def b2d(b):
        return b.reshape(1, -1)

    src = edge_index[0]
    dst = edge_index[1]
    zeros = jnp.zeros((N, H), jnp.float32)

    gather = _make_gather(N, E, H)
    scatter = _make_scatter(N, E, H)

    (nw1, nb1), (nw2, nb2) = params['node_enc']
    h = pl.pallas_call(
        _node_enc_body,
        out_shape=jax.ShapeDtypeStruct((N, H), jnp.float32),
    )(x, nw1, b2d(nb1), nw2, b2d(nb2))

    (ew1, eb1), (ew2, eb2) = params['edge_enc']

    e = None
    for k in range(len(params['edge_upd'])):
        hs, hd = gather(h, src, dst)
        (uw1, ub1), (uw2, ub2) = params['edge_upd'][k]
        if k == 0:
            e = _edge_call(
                _edge0_body, grid,
                [_rows(_BE, ED), _rows(_BE, H), _rows(_BE, H),
                 _full(ew1.shape), _full((1, H)), _full(ew2.shape),
                 _full((1, H)),
                 _full(uw1.shape), _full((1, 2 * H)), _full(uw2.shape),
                 _full((1, H))],
                _rows(_BE, H), jax.ShapeDtypeStruct((E, H), jnp.float32),
            )(edge_attr, hs, hd, ew1, b2d(eb1), ew2, b2d(eb2),
              uw1, b2d(ub1), uw2, b2d(ub2))
        else:
            e = _edge_call(
                _edgek_body, grid,
                [_rows(_BE, H), _rows(_BE, H), _rows(_BE, H),
                 _full(uw1.shape), _full((1, 2 * H)), _full(uw2.shape),
                 _full((1, H))],
                _rows(_BE, H), jax.ShapeDtypeStruct((E, H), jnp.float32),
            )(e, hs, hd, uw1, b2d(ub1), uw2, b2d(ub2))

        acc2 = scatter(e, dst, zeros)

        (vw1, vb1), (vw2, vb2) = params['node_upd'][k]
        h = pl.pallas_call(
            _node_upd_body,
            out_shape=jax.ShapeDtypeStruct((N, H), jnp.float32),
        )(h, acc2, vw1, b2d(vb1), vw2, b2d(vb2))

    hs, hd = gather(h, src, dst)
    (hw1, hb1), (hw2, hb2), (hw3, hb3) = params['edge_head']
    logits = _edge_call(
        _head_body, grid,
        [_rows(_BE, H), _rows(_BE, H), _rows(_BE, H),
         _full(hw1.shape), _full((1, 2 * H)), _full(hw2.shape),
         _full((1, H)), _full(hw3.shape), _full((1, 1))],
        _rows(_BE, 1), jax.ShapeDtypeStruct((E, 1), jnp.float32),
    )(e, hs, hd, hw1, b2d(hb1), hw2, b2d(hb2), hw3, b2d(hb3))
    return logits[:, 0]


# speedup vs baseline: 2.0818x; 1.1504x over previous
import functools

import jax
import jax.numpy as jnp
from jax import lax
from jax.experimental import pallas as pl
from jax.experimental.pallas import tpu as pltpu
from jax.experimental.pallas import tpu_sc as plsc

_NC, _NS = 2, 16
_NW = _NC * _NS
_C = 80
_G = 5


def _make_gather(N, E, D):
    per_w = E // _NW
    gedges = _G * _C
    n_groups = per_w // gedges
    mesh = plsc.VectorSubcoreMesh(core_axis_name="c", subcore_axis_name="s")

    def body(h_hbm, src_hbm, dst_hbm, hs_hbm, hd_hbm,
             idx_s, idx_d, rows_s, rows_d, sem_s, sem_d):
        wid = lax.axis_index("s") * _NC + lax.axis_index("c")
        row0 = wid * (per_w // _C)

        def group(g, carry):
            r = row0 + g * _G
            st = r * _C
            pltpu.sync_copy(src_hbm.at[pl.ds(r, _G)], idx_s)
            pltpu.sync_copy(dst_hbm.at[pl.ds(r, _G)], idx_d)
            cps = []
            for j in range(_G):
                cps.append(pltpu.async_copy(
                    h_hbm.at[idx_s.at[j]],
                    rows_s.at[pl.ds(j * _C, _C)], sem_s))
                cps.append(pltpu.async_copy(
                    h_hbm.at[idx_d.at[j]],
                    rows_d.at[pl.ds(j * _C, _C)], sem_d))
            for cp in cps:
                cp.wait()
            pltpu.sync_copy(rows_s, hs_hbm.at[pl.ds(st, gedges)])
            pltpu.sync_copy(rows_d, hd_hbm.at[pl.ds(st, gedges)])
            return carry

        lax.fori_loop(0, n_groups, group, 0)

    return pl.kernel(
        body,
        out_type=(jax.ShapeDtypeStruct((E, D), jnp.bfloat16),
                  jax.ShapeDtypeStruct((E, D), jnp.bfloat16)),
        mesh=mesh,
        scratch_types=[
            pltpu.VMEM((_G, _C), jnp.int32),
            pltpu.VMEM((_G, _C), jnp.int32),
            pltpu.VMEM((_G * _C, D), jnp.bfloat16),
            pltpu.VMEM((_G * _C, D), jnp.bfloat16),
            pltpu.SemaphoreType.DMA,
            pltpu.SemaphoreType.DMA,
        ],
        compiler_params=pltpu.CompilerParams(use_tc_tiling_on_sc=False),
    )


def _make_scatter(N, E, D):
    per_w = E // _NW
    gedges = _G * _C
    n_groups = per_w // gedges
    rows_per_s = N // _NS
    mesh = plsc.VectorSubcoreMesh(core_axis_name="c", subcore_axis_name="s")

    def body(e_hbm, dst_hbm, zeros_hbm, out_hbm, acc_sh, idx_v, rows_v, sem):
        c = lax.axis_index("c")
        s = lax.axis_index("s")
        wid = s * _NC + c
        row0 = wid * (per_w // _C)
        pltpu.sync_copy(zeros_hbm.at[pl.ds(s * rows_per_s, rows_per_s)],
                        acc_sh.at[pl.ds(s * rows_per_s, rows_per_s)])
        plsc.subcore_barrier()

        def group(g, carry):
            r = row0 + g * _G
            st = r * _C
            pltpu.sync_copy(dst_hbm.at[pl.ds(r, _G)], idx_v)
            pltpu.sync_copy(e_hbm.at[pl.ds(st, gedges)], rows_v)
            cps = [pltpu.async_copy(rows_v.at[pl.ds(j * _C, _C)],
                                    acc_sh.at[idx_v.at[j]], sem, add=True)
                   for j in range(_G)]
            for cp in cps:
                cp.wait()
            return carry

        lax.fori_loop(0, n_groups, group, 0)
        plsc.subcore_barrier()
        pltpu.sync_copy(acc_sh.at[pl.ds(s * rows_per_s, rows_per_s)],
                        out_hbm.at[c, pl.ds(s * rows_per_s, rows_per_s)])

    return pl.kernel(
        body,
        out_type=jax.ShapeDtypeStruct((_NC, N, D), jnp.float32),
        mesh=mesh,
        scratch_types=[
            pltpu.VMEM_SHARED((N, D), jnp.float32),
            pltpu.VMEM((_G, _C), jnp.int32),
            pltpu.VMEM((_G * _C, D), jnp.float32),
            pltpu.SemaphoreType.DMA,
        ],
        compiler_params=pltpu.CompilerParams(use_tc_tiling_on_sc=False),
    )


def _dot(a, b):
    return jnp.dot(a, b, preferred_element_type=jnp.float32)


def _bf(x):
    return x.astype(jnp.bfloat16)


def _node_enc_body(x_ref, w1, b1, w2, b2, o_ref):
    z = jnp.maximum(_dot(x_ref[...], w1[...]) + b1[...], 0.0)
    o_ref[...] = _bf(_dot(z, w2[...]) + b2[...])


def _node_upd_body(h_ref, acc_ref, w1, b1, w2, b2, o_ref):
    agg = acc_ref[0] + acc_ref[1]
    hin = jnp.concatenate([h_ref[...], _bf(agg)], axis=1)
    z = jnp.maximum(_dot(hin, _bf(w1[...])) + b1[...], 0.0)
    o_ref[...] = _bf(_dot(z, w2[...]) + b2[...])


def _edge0_body(ea, hs, hd, ew1, eb1, ew2, eb2, w1, b1, w2, b2, o_ref):
    e0 = _dot(jnp.maximum(_dot(ea[...], ew1[...]) + eb1[...], 0.0),
              ew2[...]) + eb2[...]
    ein = jnp.concatenate([hs[...], hd[...], _bf(e0)], axis=1)
    z = jnp.maximum(_dot(ein, _bf(w1[...])) + b1[...], 0.0)
    o_ref[...] = _dot(z, w2[...]) + b2[...]


def _edgek_body(e, hs, hd, w1, b1, w2, b2, o_ref):
    ein = jnp.concatenate([hs[...], hd[...], _bf(e[...])], axis=1)
    z = jnp.maximum(_dot(ein, _bf(w1[...])) + b1[...], 0.0)
    o_ref[...] = _dot(z, w2[...]) + b2[...]


def _head_body(e, hs, hd, w1, b1, w2, b2, w3, b3, o_ref):
    ein = jnp.concatenate([hs[...], hd[...], _bf(e[...])], axis=1)
    z = jnp.maximum(_dot(ein, _bf(w1[...])) + b1[...], 0.0)
    z2 = jnp.maximum(_dot(z, w2[...]) + b2[...], 0.0)
    o_ref[...] = _dot(z2, w3[...]) + b3[...]


def _full(shape):
    return pl.BlockSpec(shape, lambda i: (0,) * len(shape))


def _rows(be, d):
    return pl.BlockSpec((be, d), lambda i: (i, 0))


def _edge_call(body, grid, in_specs, out_specs, out_shape):
    return pl.pallas_call(
        body, grid=(grid,), in_specs=in_specs, out_specs=out_specs,
        out_shape=out_shape,
        compiler_params=pltpu.CompilerParams(
            vmem_limit_bytes=100 * 1024 * 1024))


_BE = 6400


def kernel(x, edge_attr, params, edge_index):
    N, ND = x.shape
    E, ED = edge_attr.shape
    H = params['node_enc'][-1][0].shape[1]
    assert E % (_NW * _G * _C) == 0 and N % _NS == 0 and E % _BE == 0
    grid = E // _BE

    def b2d(b):
        return b.reshape(1, -1)

    src = edge_index[0].reshape(E // _C, _C)
    dst = edge_index[1].reshape(E // _C, _C)
    zeros = jnp.zeros((N, H), jnp.float32)

    gather = _make_gather(N, E, H)
    scatter = _make_scatter(N, E, H)

    (nw1, nb1), (nw2, nb2) = params['node_enc']
    h = pl.pallas_call(
        _node_enc_body,
        out_shape=jax.ShapeDtypeStruct((N, H), jnp.bfloat16),
    )(x, nw1, b2d(nb1), nw2, b2d(nb2))

    (ew1, eb1), (ew2, eb2) = params['edge_enc']

    e = None
    for k in range(len(params['edge_upd'])):
        hs, hd = gather(h, src, dst)
        (uw1, ub1), (uw2, ub2) = params['edge_upd'][k]
        if k == 0:
            e = _edge_call(
                _edge0_body, grid,
                [_rows(_BE, ED), _rows(_BE, H), _rows(_BE, H),
                 _full(ew1.shape), _full((1, H)), _full(ew2.shape),
                 _full((1, H)),
                 _full(uw1.shape), _full((1, 2 * H)), _full(uw2.shape),
                 _full((1, H))],
                _rows(_BE, H), jax.ShapeDtypeStruct((E, H), jnp.float32),
            )(edge_attr, hs, hd, ew1, b2d(eb1), ew2, b2d(eb2),
              uw1, b2d(ub1), uw2, b2d(ub2))
        else:
            e = _edge_call(
                _edgek_body, grid,
                [_rows(_BE, H), _rows(_BE, H), _rows(_BE, H),
                 _full(uw1.shape), _full((1, 2 * H)), _full(uw2.shape),
                 _full((1, H))],
                _rows(_BE, H), jax.ShapeDtypeStruct((E, H), jnp.float32),
            )(e, hs, hd, uw1, b2d(ub1), uw2, b2d(ub2))

        acc2 = scatter(e, dst, zeros)

        (vw1, vb1), (vw2, vb2) = params['node_upd'][k]
        h = pl.pallas_call(
            _node_upd_body,
            out_shape=jax.ShapeDtypeStruct((N, H), jnp.bfloat16),
        )(h, acc2, vw1, b2d(vb1), vw2, b2d(vb2))

    hs, hd = gather(h, src, dst)
    (hw1, hb1), (hw2, hb2), (hw3, hb3) = params['edge_head']
    logits = _edge_call(
        _head_body, grid,
        [_rows(_BE, H), _rows(_BE, H), _rows(_BE, H),
         _full(hw1.shape), _full((1, 2 * H)), _full(hw2.shape),
         _full((1, H)), _full(hw3.shape), _full((1, 1))],
        _rows(_BE, 1), jax.ShapeDtypeStruct((E, 1), jnp.float32),
    )(e, hs, hd, hw1, b2d(hb1), hw2, b2d(hb2), hw3, b2d(hb3))
    return logits[:, 0]
